# Initial kernel scaffold; baseline (speedup 1.0000x reference)
import functools
import math

import jax
import jax.numpy as jnp
from jax import lax
from jax.experimental import pallas as pl
from jax.experimental.pallas import tpu as pltpu
from jax.experimental.pallas import tpu_sc as plsc

F32 = jnp.float32
I32 = jnp.int32

NUM_WORKERS = 32
SC_CHUNK = 128


def _stage1_body(cond_ref, lat_ref, WkT_ref, bk_ref, WvT_ref, bv_ref,
                 WlatT_ref, be1_ref, K_ref, V_ref, LP_ref):
    cond = cond_ref[...]
    K_ref[...] = jnp.dot(cond, WkT_ref[...],
                         preferred_element_type=F32) + bk_ref[...]
    V_ref[...] = jnp.dot(cond, WvT_ref[...],
                         preferred_element_type=F32) + bv_ref[...]
    lat = lat_ref[...]
    cols = []
    for i in range(3):
        for j in range(3):
            c = (lat[:, 3 * i + 0] * lat[:, 3 * j + 0]
                 + lat[:, 3 * i + 1] * lat[:, 3 * j + 1]
                 + lat[:, 3 * i + 2] * lat[:, 3 * j + 2])
            cols.append(c[:, None])
    cols.append(jnp.zeros((lat.shape[0], 7), F32))
    ips = jnp.concatenate(cols, axis=1)
    LP_ref[...] = jnp.dot(ips, WlatT_ref[...],
                          preferred_element_type=F32) + be1_ref[...]


def _stage2_body(h0_ref, n2g_ref, fr_ref, Kr_ref, Vr_ref, WqT_ref, bq_ref,
                 WoT_ref, bo_ref, WhiT_ref, WhjT_ref,
                 h_ref, P1_ref, P2_ref, *, bn, B, T, D):
    h0 = h0_ref[...]
    Q = jnp.dot(h0, WqT_ref[...], preferred_element_type=F32) + bq_ref[...]
    g = n2g_ref[0, 0, :]
    iota = lax.broadcasted_iota(I32, (bn, B), 1)
    sel = (g[:, None] == iota).astype(F32)
    Kn = jnp.dot(sel, Kr_ref[...], preferred_element_type=F32)
    Vn = jnp.dot(sel, Vr_ref[...], preferred_element_type=F32)
    inv = 1.0 / math.sqrt(D)
    score_cols = []
    for t in range(T):
        kt = Kn[:, t * D:(t + 1) * D]
        score_cols.append(jnp.sum(Q * kt, axis=-1, keepdims=True) * inv)
    scores = jnp.concatenate(score_cols, axis=1)
    m = jnp.max(scores, axis=-1, keepdims=True)
    p = jnp.exp(scores - m)
    attn = p / jnp.sum(p, axis=-1, keepdims=True)
    ao = jnp.zeros((bn, D), F32)
    for t in range(T):
        ao = ao + attn[:, t:t + 1] * Vn[:, t * D:(t + 1) * D]
    h = h0 + jnp.dot(ao, WoT_ref[...], preferred_element_type=F32) + bo_ref[...]
    h_ref[...] = h
    fr = fr_ref[...]
    P1_ref[:, :D] = jnp.dot(h, WhiT_ref[...], preferred_element_type=F32)
    P1_ref[:, D:] = -fr
    P2_ref[:, :D] = jnp.dot(h, WhjT_ref[...], preferred_element_type=F32)
    P2_ref[:, D:] = fr


def _sc_gather_body(P1_hbm, P2_hbm, e0_hbm, e1_hbm, z_hbm,
                    i0, i1, buf, sem0, sem1, *, per_w, nch, ch):
    c = lax.axis_index("c")
    s = lax.axis_index("s")
    wid = s * 2 + c
    base_w = wid * per_w

    def step(k, carry):
        base = base_w + k * ch
        pltpu.sync_copy(e0_hbm.at[pl.ds(base, ch)], i0)
        pltpu.sync_copy(e1_hbm.at[pl.ds(base, ch)], i1)
        g1 = pltpu.async_copy(P1_hbm.at[i0], buf, sem0)
        g1.wait()
        g2 = pltpu.async_copy(P2_hbm.at[i1], buf, sem1, add=True)
        g2.wait()
        pltpu.sync_copy(buf, z_hbm.at[pl.ds(base, ch)])
        return carry

    lax.fori_loop(0, nch, step, 0)


def _stage4_body(z_ref, eg_ref, LP_ref, WfdT_ref, We2T_ref, be2_ref, y_ref,
                 *, be, B, D):
    zin = z_ref[...]
    fd = jnp.mod(zin[:, D:], 1.0)
    eg = eg_ref[0, 0, :]
    iota = lax.broadcasted_iota(I32, (be, B), 1)
    sel = (eg[:, None] == iota).astype(F32)
    z = (zin[:, :D]
         + jnp.dot(fd, WfdT_ref[...], preferred_element_type=F32)
         + jnp.dot(sel, LP_ref[...], preferred_element_type=F32))
    u = z * jax.nn.sigmoid(z)
    v = jnp.dot(u, We2T_ref[...], preferred_element_type=F32) + be2_ref[...]
    y_ref[...] = v * jax.nn.sigmoid(v)


def _sc_scatter_body(y_hbm, e0_hbm, sums_hbm, cnts_hbm,
                     ib, yb, ones_b, zb, zb16, ssum, scnt,
                     *, per_w, nch, ch, npad, D):
    c = lax.axis_index("c")
    s = lax.axis_index("s")
    wid = s * 2 + c
    base_w = wid * per_w
    zeros16 = jnp.zeros((16,), F32)
    ones16 = jnp.ones((16,), F32)

    def fill_zb(r, carry):
        for q in range(D // 16):
            zb[r, pl.ds(q * 16, 16)] = zeros16
        zb16[r, pl.ds(0, 16)] = zeros16
        ones_b[r, pl.ds(0, 16)] = ones16
        return carry

    lax.fori_loop(0, ch, fill_zb, 0)

    rows_per_tile = npad // 16
    nzc = rows_per_tile // ch

    def zero_spmem(q, carry):
        off = s * rows_per_tile + q * ch
        pltpu.sync_copy(zb, ssum.at[pl.ds(off, ch)])
        pltpu.sync_copy(zb16, scnt.at[pl.ds(off, ch)])
        return carry

    lax.fori_loop(0, nzc, zero_spmem, 0)
    plsc.subcore_barrier()

    def step(k, carry):
        base = base_w + k * ch
        pltpu.sync_copy(e0_hbm.at[pl.ds(base, ch)], ib)
        pltpu.sync_copy(y_hbm.at[pl.ds(base, ch)], yb)
        pltpu.sync_copy(yb, ssum.at[ib], add=True)
        pltpu.sync_copy(ones_b, scnt.at[ib], add=True)
        return carry

    lax.fori_loop(0, nch, step, 0)
    plsc.subcore_barrier()

    @pl.when(s == 0)
    def _():
        pltpu.sync_copy(ssum, sums_hbm.at[c])
        pltpu.sync_copy(scnt, cnts_hbm.at[c])


def _stage6_body(h_ref, h0_ref, s0_ref, s1_ref, c0_ref, c1_ref,
                 WnhT_ref, WnaT_ref, bn1_ref, Wn2T_ref, bn2_ref, out_ref):
    cnt = c0_ref[:, :1] + c1_ref[:, :1]
    agg = (s0_ref[...] + s1_ref[...]) / jnp.maximum(cnt, 1.0)
    a = (jnp.dot(h_ref[...], WnhT_ref[...], preferred_element_type=F32)
         + jnp.dot(agg, WnaT_ref[...], preferred_element_type=F32)
         + bn1_ref[...])
    a = a * jax.nn.sigmoid(a)
    b = jnp.dot(a, Wn2T_ref[...], preferred_element_type=F32) + bn2_ref[...]
    out_ref[...] = h0_ref[...] + b * jax.nn.sigmoid(b)


def _const_spec(shape):
    nd = len(shape)
    return pl.BlockSpec(shape, lambda *_: (0,) * nd)


def kernel(node_features, cond_tokens, node2graph, frac_coords, lattices,
           edges, edge2graph, Wq, bq, Wk, bk, Wv, bv, Wo, bo,
           We1, be1, We2, be2, Wn1, bn1, Wn2, bn2):
    N, D = node_features.shape
    B, T, _ = cond_tokens.shape
    E = edges.shape[1]

    BN = 256
    BE = 2048
    npad = ((N + BN - 1) // BN) * BN
    grain = NUM_WORKERS * SC_CHUNK
    epad = ((E + grain - 1) // grain) * grain
    while epad % BE != 0:
        epad += grain
    n_nb = npad // BN
    n_eb = epad // BE
    per_w = epad // NUM_WORKERS
    nch = per_w // SC_CHUNK
    DA = D + 16

    WqT = Wq.T
    WkT = Wk.T
    WvT = Wv.T
    WoT = Wo.T
    WhiT = We1[:, :D].T
    WhjT = We1[:, D:2 * D].T
    WlatT = jnp.pad(We1[:, 2 * D:2 * D + 9].T, ((0, 7), (0, 0)))
    WfdT = jnp.pad(We1[:, 2 * D + 9:2 * D + 12].T, ((0, 13), (0, 0)))
    We2T = We2.T
    WnhT = Wn1[:, :D].T
    WnaT = Wn1[:, D:].T
    Wn2T = Wn2.T
    b_row = lambda v: v.reshape(1, -1)

    h0p = jnp.pad(node_features, ((0, npad - N), (0, 0)))
    n2g3 = jnp.pad(node2graph, (0, npad - N)).reshape(n_nb, 1, BN)
    frp = jnp.pad(frac_coords, ((0, npad - N), (0, 13)))
    lat16 = jnp.pad(lattices.reshape(B, 9), ((0, 0), (0, 7)))
    condf = cond_tokens.reshape(B * T, D)

    e0p = jnp.pad(edges[0], (0, epad - E), constant_values=N)
    e1p = jnp.pad(edges[1], (0, epad - E), constant_values=0)
    eg3 = jnp.pad(edge2graph, (0, epad - E),
                  constant_values=0).reshape(n_eb, 1, BE)

    K, V, LP = pl.pallas_call(
        _stage1_body,
        out_shape=(jax.ShapeDtypeStruct((B * T, D), F32),
                   jax.ShapeDtypeStruct((B * T, D), F32),
                   jax.ShapeDtypeStruct((B, D), F32)),
    )(condf, lat16, WkT, b_row(bk), WvT, b_row(bv), WlatT, b_row(be1))
    Kr = K.reshape(B, T * D)
    Vr = V.reshape(B, T * D)

    body2 = functools.partial(_stage2_body, bn=BN, B=B, T=T, D=D)
    h, P1a, P2a = pl.pallas_call(
        body2,
        grid=(n_nb,),
        in_specs=[
            pl.BlockSpec((BN, D), lambda i: (i, 0)),
            pl.BlockSpec((1, 1, BN), lambda i: (i, 0, 0)),
            pl.BlockSpec((BN, 16), lambda i: (i, 0)),
            _const_spec((B, T * D)),
            _const_spec((B, T * D)),
            _const_spec((D, D)),
            _const_spec((1, D)),
            _const_spec((D, D)),
            _const_spec((1, D)),
            _const_spec((D, D)),
            _const_spec((D, D)),
        ],
        out_specs=[
            pl.BlockSpec((BN, D), lambda i: (i, 0)),
            pl.BlockSpec((BN, DA), lambda i: (i, 0)),
            pl.BlockSpec((BN, DA), lambda i: (i, 0)),
        ],
        out_shape=(jax.ShapeDtypeStruct((npad, D), F32),
                   jax.ShapeDtypeStruct((npad, DA), F32),
                   jax.ShapeDtypeStruct((npad, DA), F32)),
    )(h0p, n2g3, frp, Kr, Vr, WqT, b_row(bq), WoT, b_row(bo), WhiT, WhjT)

    mesh = plsc.VectorSubcoreMesh(core_axis_name="c", subcore_axis_name="s",
                                  num_cores=2, num_subcores=16)
    gather_body = functools.partial(_sc_gather_body, per_w=per_w, nch=nch,
                                    ch=SC_CHUNK)
    z = pl.kernel(
        gather_body,
        out_type=jax.ShapeDtypeStruct((epad, DA), F32),
        mesh=mesh,
        scratch_types=[
            pltpu.VMEM((SC_CHUNK,), I32),
            pltpu.VMEM((SC_CHUNK,), I32),
            pltpu.VMEM((SC_CHUNK, DA), F32),
            pltpu.SemaphoreType.DMA,
            pltpu.SemaphoreType.DMA,
        ],
    )(P1a, P2a, e0p, e1p)

    body4 = functools.partial(_stage4_body, be=BE, B=B, D=D)
    y = pl.pallas_call(
        body4,
        grid=(n_eb,),
        in_specs=[
            pl.BlockSpec((BE, DA), lambda i: (i, 0)),
            pl.BlockSpec((1, 1, BE), lambda i: (i, 0, 0)),
            _const_spec((B, D)),
            _const_spec((16, D)),
            _const_spec((D, D)),
            _const_spec((1, D)),
        ],
        out_specs=pl.BlockSpec((BE, D), lambda i: (i, 0)),
        out_shape=jax.ShapeDtypeStruct((epad, D), F32),
    )(z, eg3, LP, WfdT, We2T, b_row(be2))

    scatter_body = functools.partial(_sc_scatter_body, per_w=per_w, nch=nch,
                                     ch=SC_CHUNK, npad=npad, D=D)
    sums, cnts = pl.kernel(
        scatter_body,
        out_type=(jax.ShapeDtypeStruct((2, npad, D), F32),
                  jax.ShapeDtypeStruct((2, npad, 16), F32)),
        mesh=mesh,
        scratch_types=[
            pltpu.VMEM((SC_CHUNK,), I32),
            pltpu.VMEM((SC_CHUNK, D), F32),
            pltpu.VMEM((SC_CHUNK, 16), F32),
            pltpu.VMEM((SC_CHUNK, D), F32),
            pltpu.VMEM((SC_CHUNK, 16), F32),
            pltpu.VMEM_SHARED((npad, D), F32),
            pltpu.VMEM_SHARED((npad, 16), F32),
        ],
    )(y, e0p)

    out = pl.pallas_call(
        _stage6_body,
        grid=(n_nb,),
        in_specs=[
            pl.BlockSpec((BN, D), lambda i: (i, 0)),
            pl.BlockSpec((BN, D), lambda i: (i, 0)),
            pl.BlockSpec((BN, D), lambda i: (i, 0)),
            pl.BlockSpec((BN, D), lambda i: (i, 0)),
            pl.BlockSpec((BN, 16), lambda i: (i, 0)),
            pl.BlockSpec((BN, 16), lambda i: (i, 0)),
            _const_spec((D, D)),
            _const_spec((D, D)),
            _const_spec((1, D)),
            _const_spec((D, D)),
            _const_spec((1, D)),
        ],
        out_specs=pl.BlockSpec((BN, D), lambda i: (i, 0)),
        out_shape=jax.ShapeDtypeStruct((npad, D), F32),
    )(h, h0p, sums[0], sums[1], cnts[0], cnts[1],
      WnhT, WnaT, b_row(bn1), Wn2T, b_row(bn2))

    return out[:N]


# speedup vs baseline: 3.2807x; 3.2807x over previous
import functools
import math

import jax
import jax.numpy as jnp
from jax import lax
from jax.experimental import pallas as pl
from jax.experimental.pallas import tpu as pltpu
from jax.experimental.pallas import tpu_sc as plsc

F32 = jnp.float32
I32 = jnp.int32

NUM_WORKERS = 32
SC_CHUNK = 64


def _stage1_body(cond_ref, lat_ref, WkT_ref, bk_ref, WvT_ref, bv_ref,
                 WlatT_ref, be1_ref, K_ref, V_ref, LP_ref):
    cond = cond_ref[...]
    K_ref[...] = jnp.dot(cond, WkT_ref[...],
                         preferred_element_type=F32) + bk_ref[...]
    V_ref[...] = jnp.dot(cond, WvT_ref[...],
                         preferred_element_type=F32) + bv_ref[...]
    lat = lat_ref[...]
    cols = []
    for i in range(3):
        for j in range(3):
            c = (lat[:, 3 * i + 0] * lat[:, 3 * j + 0]
                 + lat[:, 3 * i + 1] * lat[:, 3 * j + 1]
                 + lat[:, 3 * i + 2] * lat[:, 3 * j + 2])
            cols.append(c[:, None])
    cols.append(jnp.zeros((lat.shape[0], 7), F32))
    ips = jnp.concatenate(cols, axis=1)
    LP_ref[...] = jnp.dot(ips, WlatT_ref[...],
                          preferred_element_type=F32) + be1_ref[...]


def _stage2_body(h0_ref, n2g_ref, Kr_ref, Vr_ref, WqT_ref, bq_ref,
                 WoT_ref, bo_ref, WhiT_ref, WhjT_ref,
                 h_ref, P1_ref, P2_ref, *, bn, B, T, D):
    h0 = h0_ref[...]
    Q = jnp.dot(h0, WqT_ref[...], preferred_element_type=F32) + bq_ref[...]
    g = n2g_ref[0, 0, :]
    iota = lax.broadcasted_iota(I32, (bn, B), 1)
    sel = (g[:, None] == iota).astype(F32)
    Kn = jnp.dot(sel, Kr_ref[...], preferred_element_type=F32)
    Vn = jnp.dot(sel, Vr_ref[...], preferred_element_type=F32)
    inv = 1.0 / math.sqrt(D)
    score_cols = []
    for t in range(T):
        kt = Kn[:, t * D:(t + 1) * D]
        score_cols.append(jnp.sum(Q * kt, axis=-1, keepdims=True) * inv)
    scores = jnp.concatenate(score_cols, axis=1)
    m = jnp.max(scores, axis=-1, keepdims=True)
    p = jnp.exp(scores - m)
    attn = p / jnp.sum(p, axis=-1, keepdims=True)
    ao = jnp.zeros((bn, D), F32)
    for t in range(T):
        ao = ao + attn[:, t:t + 1] * Vn[:, t * D:(t + 1) * D]
    h = h0 + jnp.dot(ao, WoT_ref[...], preferred_element_type=F32) + bo_ref[...]
    h_ref[...] = h
    P1_ref[...] = jnp.dot(h, WhiT_ref[...], preferred_element_type=F32)
    P2_ref[...] = jnp.dot(h, WhjT_ref[...], preferred_element_type=F32)


def _sc_gather_body(P1_hbm, P2_hbm, FR_hbm, e0_hbm, e1_hbm,
                    z1_hbm, z2_hbm, f1_hbm, f2_hbm,
                    i0a, i1a, b1a, b2a, g1a, g2a,
                    i0b, i1b, b1b, b2b, g1b, g2b, sema, semb,
                    *, per_w, nch, ch):
    c = lax.axis_index("c")
    s = lax.axis_index("s")
    wid = s * 2 + c
    base_w = wid * per_w
    bufs = ((i0a, i1a, b1a, b2a, g1a, g2a, sema),
            (i0b, i1b, b1b, b2b, g1b, g2b, semb))

    def fire(k, slot):
        i0, i1, b1, b2, g1, g2, sem = bufs[slot]
        base = base_w + k * ch
        pltpu.sync_copy(e0_hbm.at[pl.ds(base, ch)], i0)
        pltpu.sync_copy(e1_hbm.at[pl.ds(base, ch)], i1)
        pltpu.async_copy(P1_hbm.at[i0], b1, sem)
        pltpu.async_copy(P2_hbm.at[i1], b2, sem)
        pltpu.async_copy(FR_hbm.at[i0], g1, sem)
        pltpu.async_copy(FR_hbm.at[i1], g2, sem)

    def drain(k, slot):
        i0, i1, b1, b2, g1, g2, sem = bufs[slot]
        base = base_w + k * ch
        pltpu.make_async_copy(P1_hbm.at[i0], b1, sem).wait()
        pltpu.make_async_copy(P2_hbm.at[i1], b2, sem).wait()
        pltpu.make_async_copy(FR_hbm.at[i0], g1, sem).wait()
        pltpu.make_async_copy(FR_hbm.at[i1], g2, sem).wait()
        pltpu.sync_copy(b1, z1_hbm.at[pl.ds(base, ch)])
        pltpu.sync_copy(b2, z2_hbm.at[pl.ds(base, ch)])
        pltpu.sync_copy(g1, f1_hbm.at[pl.ds(base, ch)])
        pltpu.sync_copy(g2, f2_hbm.at[pl.ds(base, ch)])

    npairs = nch // 2
    fire(0, 0)
    fire(1, 1)

    def step(p, carry):
        k = 2 * p
        drain(k, 0)
        fire(k + 2, 0)
        drain(k + 1, 1)
        fire(k + 3, 1)
        return carry

    lax.fori_loop(0, npairs - 1, step, 0)
    drain(nch - 2, 0)
    drain(nch - 1, 1)


def _stage4_body(z1_ref, z2_ref, f1_ref, f2_ref, eg_ref, LP_ref, WfdT_ref,
                 We2T_ref, be2_ref, y_ref, *, be, B, D):
    d = f2_ref[...] - f1_ref[...]
    fd = d + jnp.where(d < 0, 1.0, 0.0)
    eg = eg_ref[0, 0, :]
    iota = lax.broadcasted_iota(I32, (be, B), 1)
    sel = (eg[:, None] == iota).astype(F32)
    z = (z1_ref[...] + z2_ref[...]
         + jnp.dot(fd, WfdT_ref[...], preferred_element_type=F32)
         + jnp.dot(sel, LP_ref[...], preferred_element_type=F32))
    u = z * jax.nn.sigmoid(z)
    v = jnp.dot(u, We2T_ref[...], preferred_element_type=F32) + be2_ref[...]
    y_ref[...] = v * jax.nn.sigmoid(v)


def _sc_sum_body(y_hbm, e0_hbm, zero_hbm, sums_hbm,
                 ib, yb, zb, acc, *, per_w, nch, ch, npad):
    c = lax.axis_index("c")
    s = lax.axis_index("s")
    wid = s * 2 + c
    base_w = wid * per_w
    pltpu.sync_copy(zero_hbm, zb)

    rows_per_tile = npad // 16
    nzc = rows_per_tile // ch

    def zero_spmem(q, carry):
        off = s * rows_per_tile + q * ch
        pltpu.sync_copy(zb, acc.at[pl.ds(off, ch)])
        return carry

    lax.fori_loop(0, nzc, zero_spmem, 0)
    plsc.subcore_barrier()

    def step(k, carry):
        base = base_w + k * ch
        pltpu.sync_copy(e0_hbm.at[pl.ds(base, ch)], ib)
        pltpu.sync_copy(y_hbm.at[pl.ds(base, ch)], yb)
        pltpu.sync_copy(yb, acc.at[ib], add=True)
        return carry

    lax.fori_loop(0, nch, step, 0)
    plsc.subcore_barrier()

    off = s * rows_per_tile
    pltpu.sync_copy(acc.at[pl.ds(off, rows_per_tile)],
                    sums_hbm.at[c].at[pl.ds(off, rows_per_tile)])


def _sc_count_body(e0_hbm, ones_hbm, zero_hbm, cnts_hbm,
                   ib, ob, zb, acc, *, per_w, nch, ch, npad):
    c = lax.axis_index("c")
    s = lax.axis_index("s")
    wid = s * 2 + c
    base_w = wid * per_w
    pltpu.sync_copy(ones_hbm, ob)
    pltpu.sync_copy(zero_hbm, zb)

    rows_per_tile = npad // 16
    nzc = rows_per_tile // ch

    def zero_spmem(q, carry):
        off = s * rows_per_tile + q * ch
        pltpu.sync_copy(zb, acc.at[pl.ds(off, ch)])
        return carry

    lax.fori_loop(0, nzc, zero_spmem, 0)
    plsc.subcore_barrier()

    def step(k, carry):
        base = base_w + k * ch
        pltpu.sync_copy(e0_hbm.at[pl.ds(base, ch)], ib)
        pltpu.sync_copy(ob, acc.at[ib], add=True)
        return carry

    lax.fori_loop(0, nch, step, 0)
    plsc.subcore_barrier()

    off = s * rows_per_tile
    pltpu.sync_copy(acc.at[pl.ds(off, rows_per_tile)],
                    cnts_hbm.at[c].at[pl.ds(off, rows_per_tile)])


def _stage6_body(h_ref, h0_ref, s0_ref, s1_ref, c0_ref, c1_ref,
                 WnhT_ref, WnaT_ref, bn1_ref, Wn2T_ref, bn2_ref, out_ref):
    cnt = c0_ref[:, :1] + c1_ref[:, :1]
    agg = (s0_ref[...] + s1_ref[...]) / jnp.maximum(cnt, 1.0)
    a = (jnp.dot(h_ref[...], WnhT_ref[...], preferred_element_type=F32)
         + jnp.dot(agg, WnaT_ref[...], preferred_element_type=F32)
         + bn1_ref[...])
    a = a * jax.nn.sigmoid(a)
    b = jnp.dot(a, Wn2T_ref[...], preferred_element_type=F32) + bn2_ref[...]
    out_ref[...] = h0_ref[...] + b * jax.nn.sigmoid(b)


def _const_spec(shape):
    nd = len(shape)
    return pl.BlockSpec(shape, lambda *_: (0,) * nd)


def kernel(node_features, cond_tokens, node2graph, frac_coords, lattices,
           edges, edge2graph, Wq, bq, Wk, bk, Wv, bv, Wo, bo,
           We1, be1, We2, be2, Wn1, bn1, Wn2, bn2):
    N, D = node_features.shape
    B, T, _ = cond_tokens.shape
    E = edges.shape[1]

    BN = 256
    BE = 2048
    npad = ((N + BN - 1) // BN) * BN
    grain = NUM_WORKERS * SC_CHUNK * 2
    epad = ((E + grain - 1) // grain) * grain
    while epad % BE != 0:
        epad += grain
    n_nb = npad // BN
    n_eb = epad // BE
    per_w = epad // NUM_WORKERS
    nch = per_w // SC_CHUNK

    WqT = Wq.T
    WkT = Wk.T
    WvT = Wv.T
    WoT = Wo.T
    WhiT = We1[:, :D].T
    WhjT = We1[:, D:2 * D].T
    WlatT = jnp.pad(We1[:, 2 * D:2 * D + 9].T, ((0, 7), (0, 0)))
    WfdT = jnp.pad(We1[:, 2 * D + 9:2 * D + 12].T, ((0, 125), (0, 0)))
    We2T = We2.T
    WnhT = Wn1[:, :D].T
    WnaT = Wn1[:, D:].T
    Wn2T = Wn2.T
    b_row = lambda v: v.reshape(1, -1)

    h0p = jnp.pad(node_features, ((0, npad - N), (0, 0)))
    n2g3 = jnp.pad(node2graph, (0, npad - N)).reshape(n_nb, 1, BN)
    fr128 = jnp.pad(frac_coords, ((0, npad - N), (0, D - 3)))
    lat16 = jnp.pad(lattices.reshape(B, 9), ((0, 0), (0, 7)))
    zrows = jnp.zeros((SC_CHUNK, D), F32)
    orows = jnp.concatenate([jnp.ones((SC_CHUNK, 1), F32),
                             jnp.zeros((SC_CHUNK, D - 1), F32)], axis=1)
    condf = cond_tokens.reshape(B * T, D)

    e0p = jnp.pad(edges[0], (0, epad - E), constant_values=N)
    e1p = jnp.pad(edges[1], (0, epad - E), constant_values=0)
    eg3 = jnp.pad(edge2graph, (0, epad - E),
                  constant_values=0).reshape(n_eb, 1, BE)

    K, V, LP = pl.pallas_call(
        _stage1_body,
        out_shape=(jax.ShapeDtypeStruct((B * T, D), F32),
                   jax.ShapeDtypeStruct((B * T, D), F32),
                   jax.ShapeDtypeStruct((B, D), F32)),
    )(condf, lat16, WkT, b_row(bk), WvT, b_row(bv), WlatT, b_row(be1))
    Kr = K.reshape(B, T * D)
    Vr = V.reshape(B, T * D)

    body2 = functools.partial(_stage2_body, bn=BN, B=B, T=T, D=D)
    h, P1a, P2a = pl.pallas_call(
        body2,
        grid=(n_nb,),
        in_specs=[
            pl.BlockSpec((BN, D), lambda i: (i, 0)),
            pl.BlockSpec((1, 1, BN), lambda i: (i, 0, 0)),
            _const_spec((B, T * D)),
            _const_spec((B, T * D)),
            _const_spec((D, D)),
            _const_spec((1, D)),
            _const_spec((D, D)),
            _const_spec((1, D)),
            _const_spec((D, D)),
            _const_spec((D, D)),
        ],
        out_specs=[
            pl.BlockSpec((BN, D), lambda i: (i, 0)),
            pl.BlockSpec((BN, D), lambda i: (i, 0)),
            pl.BlockSpec((BN, D), lambda i: (i, 0)),
        ],
        out_shape=(jax.ShapeDtypeStruct((npad, D), F32),
                   jax.ShapeDtypeStruct((npad, D), F32),
                   jax.ShapeDtypeStruct((npad, D), F32)),
    )(h0p, n2g3, Kr, Vr, WqT, b_row(bq), WoT, b_row(bo), WhiT, WhjT)

    mesh = plsc.VectorSubcoreMesh(core_axis_name="c", subcore_axis_name="s",
                                  num_cores=2, num_subcores=16)
    gather_body = functools.partial(_sc_gather_body, per_w=per_w, nch=nch,
                                    ch=SC_CHUNK)
    z1, z2, f1, f2 = pl.kernel(
        gather_body,
        out_type=(jax.ShapeDtypeStruct((epad, D), F32),
                  jax.ShapeDtypeStruct((epad, D), F32),
                  jax.ShapeDtypeStruct((epad, D), F32),
                  jax.ShapeDtypeStruct((epad, D), F32)),
        mesh=mesh,
        scratch_types=[
            pltpu.VMEM((SC_CHUNK,), I32),
            pltpu.VMEM((SC_CHUNK,), I32),
            pltpu.VMEM((SC_CHUNK, D), F32),
            pltpu.VMEM((SC_CHUNK, D), F32),
            pltpu.VMEM((SC_CHUNK, D), F32),
            pltpu.VMEM((SC_CHUNK, D), F32),
            pltpu.VMEM((SC_CHUNK,), I32),
            pltpu.VMEM((SC_CHUNK,), I32),
            pltpu.VMEM((SC_CHUNK, D), F32),
            pltpu.VMEM((SC_CHUNK, D), F32),
            pltpu.VMEM((SC_CHUNK, D), F32),
            pltpu.VMEM((SC_CHUNK, D), F32),
            pltpu.SemaphoreType.DMA,
            pltpu.SemaphoreType.DMA,
        ],
    )(P1a, P2a, fr128, e0p, e1p)

    body4 = functools.partial(_stage4_body, be=BE, B=B, D=D)
    y = pl.pallas_call(
        body4,
        grid=(n_eb,),
        in_specs=[
            pl.BlockSpec((BE, D), lambda i: (i, 0)),
            pl.BlockSpec((BE, D), lambda i: (i, 0)),
            pl.BlockSpec((BE, D), lambda i: (i, 0)),
            pl.BlockSpec((BE, D), lambda i: (i, 0)),
            pl.BlockSpec((1, 1, BE), lambda i: (i, 0, 0)),
            _const_spec((B, D)),
            _const_spec((D, D)),
            _const_spec((D, D)),
            _const_spec((1, D)),
        ],
        out_specs=pl.BlockSpec((BE, D), lambda i: (i, 0)),
        out_shape=jax.ShapeDtypeStruct((epad, D), F32),
    )(z1, z2, f1, f2, eg3, LP, WfdT, We2T, b_row(be2))

    sum_body = functools.partial(_sc_sum_body, per_w=per_w, nch=nch,
                                 ch=SC_CHUNK, npad=npad)
    sums = pl.kernel(
        sum_body,
        out_type=jax.ShapeDtypeStruct((2, npad, D), F32),
        mesh=mesh,
        scratch_types=[
            pltpu.VMEM((SC_CHUNK,), I32),
            pltpu.VMEM((SC_CHUNK, D), F32),
            pltpu.VMEM((SC_CHUNK, D), F32),
            pltpu.VMEM_SHARED((npad, D), F32),
        ],
    )(y, e0p, zrows)

    count_body = functools.partial(_sc_count_body, per_w=per_w, nch=nch,
                                   ch=SC_CHUNK, npad=npad)
    cnts = pl.kernel(
        count_body,
        out_type=jax.ShapeDtypeStruct((2, npad, D), F32),
        mesh=mesh,
        scratch_types=[
            pltpu.VMEM((SC_CHUNK,), I32),
            pltpu.VMEM((SC_CHUNK, D), F32),
            pltpu.VMEM((SC_CHUNK, D), F32),
            pltpu.VMEM_SHARED((npad, D), F32),
        ],
    )(e0p, orows, zrows)

    out = pl.pallas_call(
        _stage6_body,
        grid=(n_nb,),
        in_specs=[
            pl.BlockSpec((BN, D), lambda i: (i, 0)),
            pl.BlockSpec((BN, D), lambda i: (i, 0)),
            pl.BlockSpec((BN, D), lambda i: (i, 0)),
            pl.BlockSpec((BN, D), lambda i: (i, 0)),
            pl.BlockSpec((BN, D), lambda i: (i, 0)),
            pl.BlockSpec((BN, D), lambda i: (i, 0)),
            _const_spec((D, D)),
            _const_spec((D, D)),
            _const_spec((1, D)),
            _const_spec((D, D)),
            _const_spec((1, D)),
        ],
        out_specs=pl.BlockSpec((BN, D), lambda i: (i, 0)),
        out_shape=jax.ShapeDtypeStruct((npad, D), F32),
    )(h, h0p, sums[0], sums[1], cnts[0], cnts[1],
      WnhT, WnaT, b_row(bn1), Wn2T, b_row(bn2))

    return out[:N]


# speedup vs baseline: 3.6013x; 1.0977x over previous
import functools
import math

import jax
import jax.numpy as jnp
from jax import lax
from jax.experimental import pallas as pl
from jax.experimental.pallas import tpu as pltpu
from jax.experimental.pallas import tpu_sc as plsc

F32 = jnp.float32
I32 = jnp.int32

NUM_WORKERS = 32
SC_CHUNK = 64


def _stage1_body(cond_ref, lat_ref, WkT_ref, bk_ref, WvT_ref, bv_ref,
                 WlatT_ref, be1_ref, K_ref, V_ref, LP_ref):
    cond = cond_ref[...]
    K_ref[...] = jnp.dot(cond, WkT_ref[...],
                         preferred_element_type=F32) + bk_ref[...]
    V_ref[...] = jnp.dot(cond, WvT_ref[...],
                         preferred_element_type=F32) + bv_ref[...]
    lat = lat_ref[...]
    cols = []
    for i in range(3):
        for j in range(3):
            c = (lat[:, 3 * i + 0] * lat[:, 3 * j + 0]
                 + lat[:, 3 * i + 1] * lat[:, 3 * j + 1]
                 + lat[:, 3 * i + 2] * lat[:, 3 * j + 2])
            cols.append(c[:, None])
    cols.append(jnp.zeros((lat.shape[0], 7), F32))
    ips = jnp.concatenate(cols, axis=1)
    LP_ref[...] = jnp.dot(ips, WlatT_ref[...],
                          preferred_element_type=F32) + be1_ref[...]


def _stage2_body(h0_ref, n2g_ref, Kr_ref, Vr_ref, WqT_ref, bq_ref,
                 WoT_ref, bo_ref, WhiT_ref, WhjT_ref,
                 h_ref, P1_ref, P2_ref, *, bn, B, T, D):
    h0 = h0_ref[...]
    Q = jnp.dot(h0, WqT_ref[...], preferred_element_type=F32) + bq_ref[...]
    g = n2g_ref[0, 0, :]
    iota = lax.broadcasted_iota(I32, (bn, B), 1)
    sel = (g[:, None] == iota).astype(F32)
    Kn = jnp.dot(sel, Kr_ref[...], preferred_element_type=F32)
    Vn = jnp.dot(sel, Vr_ref[...], preferred_element_type=F32)
    inv = 1.0 / math.sqrt(D)
    score_cols = []
    for t in range(T):
        kt = Kn[:, t * D:(t + 1) * D]
        score_cols.append(jnp.sum(Q * kt, axis=-1, keepdims=True) * inv)
    scores = jnp.concatenate(score_cols, axis=1)
    m = jnp.max(scores, axis=-1, keepdims=True)
    p = jnp.exp(scores - m)
    attn = p / jnp.sum(p, axis=-1, keepdims=True)
    ao = jnp.zeros((bn, D), F32)
    for t in range(T):
        ao = ao + attn[:, t:t + 1] * Vn[:, t * D:(t + 1) * D]
    h = h0 + jnp.dot(ao, WoT_ref[...], preferred_element_type=F32) + bo_ref[...]
    h_ref[...] = h
    P1_ref[...] = jnp.dot(h, WhiT_ref[...], preferred_element_type=F32)
    P2_ref[...] = jnp.dot(h, WhjT_ref[...], preferred_element_type=F32)


def _sc_gather_body(P1_hbm, P2_hbm, FR_hbm, e0_hbm, e1_hbm,
                    z1_hbm, z2_hbm, f1_hbm, f2_hbm,
                    i0a, i1a, b1a, b2a, g1a, g2a,
                    i0b, i1b, b1b, b2b, g1b, g2b, sema, semb,
                    *, per_w, nch, ch):
    c = lax.axis_index("c")
    s = lax.axis_index("s")
    wid = s * 2 + c
    base_w = wid * per_w
    bufs = ((i0a, i1a, b1a, b2a, g1a, g2a, sema),
            (i0b, i1b, b1b, b2b, g1b, g2b, semb))

    def fire(k, slot):
        i0, i1, b1, b2, g1, g2, sem = bufs[slot]
        base = base_w + k * ch
        pltpu.sync_copy(e0_hbm.at[pl.ds(base, ch)], i0)
        pltpu.sync_copy(e1_hbm.at[pl.ds(base, ch)], i1)
        pltpu.async_copy(P1_hbm.at[i0], b1, sem)
        pltpu.async_copy(P2_hbm.at[i1], b2, sem)
        pltpu.async_copy(FR_hbm.at[i0], g1, sem)
        pltpu.async_copy(FR_hbm.at[i1], g2, sem)

    def drain(k, slot):
        i0, i1, b1, b2, g1, g2, sem = bufs[slot]
        base = base_w + k * ch
        pltpu.make_async_copy(P1_hbm.at[i0], b1, sem).wait()
        pltpu.make_async_copy(P2_hbm.at[i1], b2, sem).wait()
        pltpu.make_async_copy(FR_hbm.at[i0], g1, sem).wait()
        pltpu.make_async_copy(FR_hbm.at[i1], g2, sem).wait()
        pltpu.sync_copy(b1, z1_hbm.at[pl.ds(base, ch)])
        pltpu.sync_copy(b2, z2_hbm.at[pl.ds(base, ch)])
        pltpu.sync_copy(g1, f1_hbm.at[pl.ds(base, ch)])
        pltpu.sync_copy(g2, f2_hbm.at[pl.ds(base, ch)])

    npairs = nch // 2
    fire(0, 0)
    fire(1, 1)

    def step(p, carry):
        k = 2 * p
        drain(k, 0)
        fire(k + 2, 0)
        drain(k + 1, 1)
        fire(k + 3, 1)
        return carry

    lax.fori_loop(0, npairs - 1, step, 0)
    drain(nch - 2, 0)
    drain(nch - 1, 1)


def _sc_count_body(e0_hbm, ones_hbm, zero_hbm, cnts_hbm,
                   ib, ob, zb, acc, *, per_w, nch, ch, npad):
    c = lax.axis_index("c")
    s = lax.axis_index("s")
    wid = s * 2 + c
    base_w = wid * per_w
    pltpu.sync_copy(ones_hbm, ob)
    pltpu.sync_copy(zero_hbm, zb)

    rows_per_tile = npad // 16
    nzc = rows_per_tile // ch

    def zero_spmem(q, carry):
        off = s * rows_per_tile + q * ch
        pltpu.sync_copy(zb, acc.at[pl.ds(off, ch)])
        return carry

    lax.fori_loop(0, nzc, zero_spmem, 0)
    plsc.subcore_barrier()

    def step(k, carry):
        base = base_w + k * ch
        pltpu.sync_copy(e0_hbm.at[pl.ds(base, ch)], ib)
        pltpu.sync_copy(ob, acc.at[ib], add=True)
        return carry

    lax.fori_loop(0, nch, step, 0)
    plsc.subcore_barrier()

    off = s * rows_per_tile
    pltpu.sync_copy(acc.at[pl.ds(off, rows_per_tile)],
                    cnts_hbm.at[c].at[pl.ds(off, rows_per_tile)])


def _stage4_body(z1_ref, z2_ref, f1_ref, f2_ref, eg_ref, LP_ref, WfdT_ref,
                 We2T_ref, be2_ref, y_ref, *, be, B, D):
    d = f2_ref[...] - f1_ref[...]
    fd = d + jnp.where(d < 0, 1.0, 0.0)
    eg = eg_ref[0, 0, :]
    iota = lax.broadcasted_iota(I32, (be, B), 1)
    sel = (eg[:, None] == iota).astype(F32)
    z = (z1_ref[...] + z2_ref[...]
         + jnp.dot(fd, WfdT_ref[...], preferred_element_type=F32)
         + jnp.dot(sel, LP_ref[...], preferred_element_type=F32))
    u = z * jax.nn.sigmoid(z)
    v = jnp.dot(u, We2T_ref[...], preferred_element_type=F32) + be2_ref[...]
    y_ref[...] = v * jax.nn.sigmoid(v)


def _sc_sum_body(y_hbm, e0_hbm, zero_hbm, sums_hbm,
                 iba, yba, ibb, ybb, zb, acc, sema, semb,
                 *, per_w, nch, ch, npad):
    c = lax.axis_index("c")
    s = lax.axis_index("s")
    wid = s * 2 + c
    base_w = wid * per_w
    pltpu.sync_copy(zero_hbm, zb)

    rows_per_tile = npad // 16
    nzc = rows_per_tile // ch

    def zero_spmem(q, carry):
        off = s * rows_per_tile + q * ch
        pltpu.sync_copy(zb, acc.at[pl.ds(off, ch)])
        return carry

    lax.fori_loop(0, nzc, zero_spmem, 0)
    plsc.subcore_barrier()

    bufs = ((iba, yba, sema), (ibb, ybb, semb))

    def fire(k, slot):
        ib, yb, sem = bufs[slot]
        base = base_w + k * ch
        pltpu.sync_copy(e0_hbm.at[pl.ds(base, ch)], ib)
        pltpu.async_copy(y_hbm.at[pl.ds(base, ch)], yb, sem)

    def drain(k, slot):
        ib, yb, sem = bufs[slot]
        base = base_w + k * ch
        pltpu.make_async_copy(y_hbm.at[pl.ds(base, ch)], yb, sem).wait()
        pltpu.sync_copy(yb, acc.at[ib], add=True)

    npairs = nch // 2
    fire(0, 0)
    fire(1, 1)

    def step(p, carry):
        k = 2 * p
        drain(k, 0)
        fire(k + 2, 0)
        drain(k + 1, 1)
        fire(k + 3, 1)
        return carry

    lax.fori_loop(0, npairs - 1, step, 0)
    drain(nch - 2, 0)
    drain(nch - 1, 1)
    plsc.subcore_barrier()

    off = s * rows_per_tile
    pltpu.sync_copy(acc.at[pl.ds(off, rows_per_tile)],
                    sums_hbm.at[c].at[pl.ds(off, rows_per_tile)])


def _stage6_body(h_ref, h0_ref, s0_ref, s1_ref, c0_ref, c1_ref,
                 WnhT_ref, WnaT_ref, bn1_ref, Wn2T_ref, bn2_ref, out_ref):
    cnt = c0_ref[:, :1] + c1_ref[:, :1]
    agg = (s0_ref[...] + s1_ref[...]) / jnp.maximum(cnt, 1.0)
    a = (jnp.dot(h_ref[...], WnhT_ref[...], preferred_element_type=F32)
         + jnp.dot(agg, WnaT_ref[...], preferred_element_type=F32)
         + bn1_ref[...])
    a = a * jax.nn.sigmoid(a)
    b = jnp.dot(a, Wn2T_ref[...], preferred_element_type=F32) + bn2_ref[...]
    out_ref[...] = h0_ref[...] + b * jax.nn.sigmoid(b)


def _const_spec(shape):
    nd = len(shape)
    return pl.BlockSpec(shape, lambda *_: (0,) * nd)


def kernel(node_features, cond_tokens, node2graph, frac_coords, lattices,
           edges, edge2graph, Wq, bq, Wk, bk, Wv, bv, Wo, bo,
           We1, be1, We2, be2, Wn1, bn1, Wn2, bn2):
    N, D = node_features.shape
    B, T, _ = cond_tokens.shape
    E = edges.shape[1]

    BN = 256
    BE = 2048
    npad = ((N + BN - 1) // BN) * BN
    grain = NUM_WORKERS * SC_CHUNK * 2
    epad = ((E + grain - 1) // grain) * grain
    while epad % BE != 0:
        epad += grain
    n_nb = npad // BN
    n_eb = epad // BE
    per_w = epad // NUM_WORKERS
    nch = per_w // SC_CHUNK

    WqT = Wq.T
    WkT = Wk.T
    WvT = Wv.T
    WoT = Wo.T
    WhiT = We1[:, :D].T
    WhjT = We1[:, D:2 * D].T
    WlatT = jnp.pad(We1[:, 2 * D:2 * D + 9].T, ((0, 7), (0, 0)))
    WfdT = jnp.pad(We1[:, 2 * D + 9:2 * D + 12].T, ((0, 125), (0, 0)))
    We2T = We2.T
    WnhT = Wn1[:, :D].T
    WnaT = Wn1[:, D:].T
    Wn2T = Wn2.T
    b_row = lambda v: v.reshape(1, -1)

    h0p = jnp.pad(node_features, ((0, npad - N), (0, 0)))
    n2g3 = jnp.pad(node2graph, (0, npad - N)).reshape(n_nb, 1, BN)
    fr128 = jnp.pad(frac_coords, ((0, npad - N), (0, D - 3)))
    lat16 = jnp.pad(lattices.reshape(B, 9), ((0, 0), (0, 7)))
    zrows = jnp.zeros((SC_CHUNK, D), F32)
    orows = jnp.concatenate([jnp.ones((SC_CHUNK, 1), F32),
                             jnp.zeros((SC_CHUNK, D - 1), F32)], axis=1)
    condf = cond_tokens.reshape(B * T, D)

    e0p = jnp.pad(edges[0], (0, epad - E), constant_values=N)
    e1p = jnp.pad(edges[1], (0, epad - E), constant_values=0)
    eg3 = jnp.pad(edge2graph, (0, epad - E),
                  constant_values=0).reshape(n_eb, 1, BE)

    K, V, LP = pl.pallas_call(
        _stage1_body,
        out_shape=(jax.ShapeDtypeStruct((B * T, D), F32),
                   jax.ShapeDtypeStruct((B * T, D), F32),
                   jax.ShapeDtypeStruct((B, D), F32)),
    )(condf, lat16, WkT, b_row(bk), WvT, b_row(bv), WlatT, b_row(be1))
    Kr = K.reshape(B, T * D)
    Vr = V.reshape(B, T * D)

    body2 = functools.partial(_stage2_body, bn=BN, B=B, T=T, D=D)
    h, P1a, P2a = pl.pallas_call(
        body2,
        grid=(n_nb,),
        in_specs=[
            pl.BlockSpec((BN, D), lambda i: (i, 0)),
            pl.BlockSpec((1, 1, BN), lambda i: (i, 0, 0)),
            _const_spec((B, T * D)),
            _const_spec((B, T * D)),
            _const_spec((D, D)),
            _const_spec((1, D)),
            _const_spec((D, D)),
            _const_spec((1, D)),
            _const_spec((D, D)),
            _const_spec((D, D)),
        ],
        out_specs=[
            pl.BlockSpec((BN, D), lambda i: (i, 0)),
            pl.BlockSpec((BN, D), lambda i: (i, 0)),
            pl.BlockSpec((BN, D), lambda i: (i, 0)),
        ],
        out_shape=(jax.ShapeDtypeStruct((npad, D), F32),
                   jax.ShapeDtypeStruct((npad, D), F32),
                   jax.ShapeDtypeStruct((npad, D), F32)),
    )(h0p, n2g3, Kr, Vr, WqT, b_row(bq), WoT, b_row(bo), WhiT, WhjT)

    mesh = plsc.VectorSubcoreMesh(core_axis_name="c", subcore_axis_name="s",
                                  num_cores=2, num_subcores=16)
    count_body = functools.partial(_sc_count_body, per_w=per_w, nch=nch,
                                   ch=SC_CHUNK, npad=npad)
    cnts = pl.kernel(
        count_body,
        out_type=jax.ShapeDtypeStruct((2, npad, D), F32),
        mesh=mesh,
        scratch_types=[
            pltpu.VMEM((SC_CHUNK,), I32),
            pltpu.VMEM((SC_CHUNK, D), F32),
            pltpu.VMEM((SC_CHUNK, D), F32),
            pltpu.VMEM_SHARED((npad, D), F32),
        ],
    )(e0p, orows, zrows)

    gather_body = functools.partial(_sc_gather_body, per_w=per_w, nch=nch,
                                    ch=SC_CHUNK)
    z1, z2, f1, f2 = pl.kernel(
        gather_body,
        out_type=(jax.ShapeDtypeStruct((epad, D), F32),
                  jax.ShapeDtypeStruct((epad, D), F32),
                  jax.ShapeDtypeStruct((epad, D), F32),
                  jax.ShapeDtypeStruct((epad, D), F32)),
        mesh=mesh,
        scratch_types=[
            pltpu.VMEM((SC_CHUNK,), I32),
            pltpu.VMEM((SC_CHUNK,), I32),
            pltpu.VMEM((SC_CHUNK, D), F32),
            pltpu.VMEM((SC_CHUNK, D), F32),
            pltpu.VMEM((SC_CHUNK, D), F32),
            pltpu.VMEM((SC_CHUNK, D), F32),
            pltpu.VMEM((SC_CHUNK,), I32),
            pltpu.VMEM((SC_CHUNK,), I32),
            pltpu.VMEM((SC_CHUNK, D), F32),
            pltpu.VMEM((SC_CHUNK, D), F32),
            pltpu.VMEM((SC_CHUNK, D), F32),
            pltpu.VMEM((SC_CHUNK, D), F32),
            pltpu.SemaphoreType.DMA,
            pltpu.SemaphoreType.DMA,
        ],
    )(P1a, P2a, fr128, e0p, e1p)

    body4 = functools.partial(_stage4_body, be=BE, B=B, D=D)
    y = pl.pallas_call(
        body4,
        grid=(n_eb,),
        in_specs=[
            pl.BlockSpec((BE, D), lambda i: (i, 0)),
            pl.BlockSpec((BE, D), lambda i: (i, 0)),
            pl.BlockSpec((BE, D), lambda i: (i, 0)),
            pl.BlockSpec((BE, D), lambda i: (i, 0)),
            pl.BlockSpec((1, 1, BE), lambda i: (i, 0, 0)),
            _const_spec((B, D)),
            _const_spec((D, D)),
            _const_spec((D, D)),
            _const_spec((1, D)),
        ],
        out_specs=pl.BlockSpec((BE, D), lambda i: (i, 0)),
        out_shape=jax.ShapeDtypeStruct((epad, D), F32),
    )(z1, z2, f1, f2, eg3, LP, WfdT, We2T, b_row(be2))

    sum_body = functools.partial(_sc_sum_body, per_w=per_w, nch=nch,
                                 ch=SC_CHUNK, npad=npad)
    sums = pl.kernel(
        sum_body,
        out_type=jax.ShapeDtypeStruct((2, npad, D), F32),
        mesh=mesh,
        scratch_types=[
            pltpu.VMEM((SC_CHUNK,), I32),
            pltpu.VMEM((SC_CHUNK, D), F32),
            pltpu.VMEM((SC_CHUNK,), I32),
            pltpu.VMEM((SC_CHUNK, D), F32),
            pltpu.VMEM((SC_CHUNK, D), F32),
            pltpu.VMEM_SHARED((npad, D), F32),
            pltpu.SemaphoreType.DMA,
            pltpu.SemaphoreType.DMA,
        ],
    )(y, e0p, zrows)

    out = pl.pallas_call(
        _stage6_body,
        grid=(n_nb,),
        in_specs=[
            pl.BlockSpec((BN, D), lambda i: (i, 0)),
            pl.BlockSpec((BN, D), lambda i: (i, 0)),
            pl.BlockSpec((BN, D), lambda i: (i, 0)),
            pl.BlockSpec((BN, D), lambda i: (i, 0)),
            pl.BlockSpec((BN, D), lambda i: (i, 0)),
            pl.BlockSpec((BN, D), lambda i: (i, 0)),
            _const_spec((D, D)),
            _const_spec((D, D)),
            _const_spec((1, D)),
            _const_spec((D, D)),
            _const_spec((1, D)),
        ],
        out_specs=pl.BlockSpec((BN, D), lambda i: (i, 0)),
        out_shape=jax.ShapeDtypeStruct((npad, D), F32),
    )(h, h0p, sums[0], sums[1], cnts[0], cnts[1],
      WnhT, WnaT, b_row(bn1), Wn2T, b_row(bn2))

    return out[:N]


# speedup vs baseline: 3.7785x; 1.0492x over previous
import functools
import math

import jax
import jax.numpy as jnp
from jax import lax
from jax.experimental import pallas as pl
from jax.experimental.pallas import tpu as pltpu
from jax.experimental.pallas import tpu_sc as plsc

F32 = jnp.float32
I32 = jnp.int32

NUM_WORKERS = 32
SC_CHUNK = 64


def _stage1_body(cond_ref, lat_ref, WkT_ref, bk_ref, WvT_ref, bv_ref,
                 WlatT_ref, be1_ref, K_ref, V_ref, LP_ref):
    cond = cond_ref[...]
    K_ref[...] = jnp.dot(cond, WkT_ref[...],
                         preferred_element_type=F32) + bk_ref[...]
    V_ref[...] = jnp.dot(cond, WvT_ref[...],
                         preferred_element_type=F32) + bv_ref[...]
    lat = lat_ref[...]
    cols = []
    for i in range(3):
        for j in range(3):
            c = (lat[:, 3 * i + 0] * lat[:, 3 * j + 0]
                 + lat[:, 3 * i + 1] * lat[:, 3 * j + 1]
                 + lat[:, 3 * i + 2] * lat[:, 3 * j + 2])
            cols.append(c[:, None])
    cols.append(jnp.zeros((lat.shape[0], 7), F32))
    ips = jnp.concatenate(cols, axis=1)
    LP_ref[...] = jnp.dot(ips, WlatT_ref[...],
                          preferred_element_type=F32) + be1_ref[...]


def _stage2_body(h0_ref, n2g_ref, Kr_ref, Vr_ref, WqT_ref, bq_ref,
                 WoT_ref, bo_ref, WhiT_ref, WhjT_ref,
                 h_ref, P1_ref, P2_ref, *, bn, B, T, D):
    h0 = h0_ref[...]
    Q = jnp.dot(h0, WqT_ref[...], preferred_element_type=F32) + bq_ref[...]
    g = n2g_ref[0, 0, :]
    iota = lax.broadcasted_iota(I32, (bn, B), 1)
    sel = (g[:, None] == iota).astype(F32)
    Kn = jnp.dot(sel, Kr_ref[...], preferred_element_type=F32)
    Vn = jnp.dot(sel, Vr_ref[...], preferred_element_type=F32)
    inv = 1.0 / math.sqrt(D)
    score_cols = []
    for t in range(T):
        kt = Kn[:, t * D:(t + 1) * D]
        score_cols.append(jnp.sum(Q * kt, axis=-1, keepdims=True) * inv)
    scores = jnp.concatenate(score_cols, axis=1)
    m = jnp.max(scores, axis=-1, keepdims=True)
    p = jnp.exp(scores - m)
    attn = p / jnp.sum(p, axis=-1, keepdims=True)
    ao = jnp.zeros((bn, D), F32)
    for t in range(T):
        ao = ao + attn[:, t:t + 1] * Vn[:, t * D:(t + 1) * D]
    h = h0 + jnp.dot(ao, WoT_ref[...], preferred_element_type=F32) + bo_ref[...]
    h_ref[...] = h
    P1_ref[...] = jnp.dot(h, WhiT_ref[...], preferred_element_type=F32)
    P2_ref[...] = jnp.dot(h, WhjT_ref[...], preferred_element_type=F32)


def _sc_gather_body(P1_hbm, P2_hbm, FR_hbm, e0_hbm, e1_hbm,
                    z1_hbm, z2_hbm, fd_hbm,
                    i0a, i1a, b1a, b2a, g1a, g2a,
                    i0b, i1b, b1b, b2b, g1b, g2b, fdbuf, sema, semb,
                    *, per_w, nch, ch):
    c = lax.axis_index("c")
    s = lax.axis_index("s")
    wid = s * 2 + c
    base_w = wid * per_w
    bufs = ((i0a, i1a, b1a, b2a, g1a, g2a, sema),
            (i0b, i1b, b1b, b2b, g1b, g2b, semb))

    def fire(k, slot):
        i0, i1, b1, b2, g1, g2, sem = bufs[slot]
        base = base_w + k * ch
        pltpu.sync_copy(e0_hbm.at[pl.ds(base, ch)], i0)
        pltpu.sync_copy(e1_hbm.at[pl.ds(base, ch)], i1)
        pltpu.async_copy(P1_hbm.at[i0], b1, sem)
        pltpu.async_copy(P2_hbm.at[i1], b2, sem)
        pltpu.async_copy(FR_hbm.at[i0], g1, sem)
        pltpu.async_copy(FR_hbm.at[i1], g2, sem)

    def drain(k, slot):
        i0, i1, b1, b2, g1, g2, sem = bufs[slot]
        base = base_w + k * ch
        pltpu.make_async_copy(P1_hbm.at[i0], b1, sem).wait()
        pltpu.make_async_copy(P2_hbm.at[i1], b2, sem).wait()
        pltpu.make_async_copy(FR_hbm.at[i0], g1, sem).wait()
        pltpu.make_async_copy(FR_hbm.at[i1], g2, sem).wait()
        pltpu.sync_copy(b1, z1_hbm.at[pl.ds(base, ch)])
        pltpu.sync_copy(b2, z2_hbm.at[pl.ds(base, ch)])

        def frac_diff(r, carry):
            fdbuf[r, pl.ds(0, 16)] = (g2[r, pl.ds(0, 16)]
                                      - g1[r, pl.ds(0, 16)])
            return carry

        lax.fori_loop(0, ch, frac_diff, 0)
        pltpu.sync_copy(fdbuf, fd_hbm.at[pl.ds(base, ch)])

    npairs = nch // 2
    fire(0, 0)
    fire(1, 1)

    def step(p, carry):
        k = 2 * p
        drain(k, 0)
        fire(k + 2, 0)
        drain(k + 1, 1)
        fire(k + 3, 1)
        return carry

    lax.fori_loop(0, npairs - 1, step, 0)
    drain(nch - 2, 0)
    drain(nch - 1, 1)


def _sc_count_body(e0_hbm, ones_hbm, zero_hbm, cnts_hbm,
                   ib, ob, zb, acc, *, per_w, nch, ch, npad):
    c = lax.axis_index("c")
    s = lax.axis_index("s")
    wid = s * 2 + c
    base_w = wid * per_w
    pltpu.sync_copy(ones_hbm, ob)
    pltpu.sync_copy(zero_hbm, zb)

    rows_per_tile = npad // 16
    nzc = rows_per_tile // ch

    def zero_spmem(q, carry):
        off = s * rows_per_tile + q * ch
        pltpu.sync_copy(zb, acc.at[pl.ds(off, ch)])
        return carry

    lax.fori_loop(0, nzc, zero_spmem, 0)
    plsc.subcore_barrier()

    def step(k, carry):
        base = base_w + k * ch
        pltpu.sync_copy(e0_hbm.at[pl.ds(base, ch)], ib)
        pltpu.sync_copy(ob, acc.at[ib], add=True)
        return carry

    lax.fori_loop(0, nch, step, 0)
    plsc.subcore_barrier()

    off = s * rows_per_tile
    pltpu.sync_copy(acc.at[pl.ds(off, rows_per_tile)],
                    cnts_hbm.at[c].at[pl.ds(off, rows_per_tile)])


def _stage4_body(z1_ref, z2_ref, fd_ref, eg_ref, LP_ref, WfdT_ref,
                 We2T_ref, be2_ref, y_ref, *, be, B, D):
    d = fd_ref[...]
    fd = d + jnp.where(d < 0, 1.0, 0.0)
    eg = eg_ref[0, 0, :]
    iota = lax.broadcasted_iota(I32, (be, B), 1)
    sel = (eg[:, None] == iota).astype(F32)
    z = (z1_ref[...] + z2_ref[...]
         + jnp.dot(fd, WfdT_ref[...], preferred_element_type=F32)
         + jnp.dot(sel, LP_ref[...], preferred_element_type=F32))
    u = z * jax.nn.sigmoid(z)
    v = jnp.dot(u, We2T_ref[...], preferred_element_type=F32) + be2_ref[...]
    y_ref[...] = v * jax.nn.sigmoid(v)


def _sc_sum_body(y_hbm, e0_hbm, zero_hbm, sums_hbm,
                 iba, yba, ibb, ybb, zb, acc, sema, semb,
                 *, per_w, nch, ch, npad):
    c = lax.axis_index("c")
    s = lax.axis_index("s")
    wid = s * 2 + c
    base_w = wid * per_w
    pltpu.sync_copy(zero_hbm, zb)

    rows_per_tile = npad // 16
    nzc = rows_per_tile // ch

    def zero_spmem(q, carry):
        off = s * rows_per_tile + q * ch
        pltpu.sync_copy(zb, acc.at[pl.ds(off, ch)])
        return carry

    lax.fori_loop(0, nzc, zero_spmem, 0)
    plsc.subcore_barrier()

    bufs = ((iba, yba, sema), (ibb, ybb, semb))

    def fire(k, slot):
        ib, yb, sem = bufs[slot]
        base = base_w + k * ch
        pltpu.sync_copy(e0_hbm.at[pl.ds(base, ch)], ib)
        pltpu.async_copy(y_hbm.at[pl.ds(base, ch)], yb, sem)

    def drain(k, slot):
        ib, yb, sem = bufs[slot]
        base = base_w + k * ch
        pltpu.make_async_copy(y_hbm.at[pl.ds(base, ch)], yb, sem).wait()
        pltpu.sync_copy(yb, acc.at[ib], add=True)

    npairs = nch // 2
    fire(0, 0)
    fire(1, 1)

    def step(p, carry):
        k = 2 * p
        drain(k, 0)
        fire(k + 2, 0)
        drain(k + 1, 1)
        fire(k + 3, 1)
        return carry

    lax.fori_loop(0, npairs - 1, step, 0)
    drain(nch - 2, 0)
    drain(nch - 1, 1)
    plsc.subcore_barrier()

    off = s * rows_per_tile
    pltpu.sync_copy(acc.at[pl.ds(off, rows_per_tile)],
                    sums_hbm.at[c].at[pl.ds(off, rows_per_tile)])


def _stage6_body(h_ref, h0_ref, s0_ref, s1_ref, c0_ref, c1_ref,
                 WnhT_ref, WnaT_ref, bn1_ref, Wn2T_ref, bn2_ref, out_ref):
    cnt = c0_ref[:, :1] + c1_ref[:, :1]
    agg = (s0_ref[...] + s1_ref[...]) / jnp.maximum(cnt, 1.0)
    a = (jnp.dot(h_ref[...], WnhT_ref[...], preferred_element_type=F32)
         + jnp.dot(agg, WnaT_ref[...], preferred_element_type=F32)
         + bn1_ref[...])
    a = a * jax.nn.sigmoid(a)
    b = jnp.dot(a, Wn2T_ref[...], preferred_element_type=F32) + bn2_ref[...]
    out_ref[...] = h0_ref[...] + b * jax.nn.sigmoid(b)


def _const_spec(shape):
    nd = len(shape)
    return pl.BlockSpec(shape, lambda *_: (0,) * nd)


def kernel(node_features, cond_tokens, node2graph, frac_coords, lattices,
           edges, edge2graph, Wq, bq, Wk, bk, Wv, bv, Wo, bo,
           We1, be1, We2, be2, Wn1, bn1, Wn2, bn2):
    N, D = node_features.shape
    B, T, _ = cond_tokens.shape
    E = edges.shape[1]

    BN = 256
    BE = 2048
    npad = ((N + BN - 1) // BN) * BN
    grain = NUM_WORKERS * SC_CHUNK * 2
    epad = ((E + grain - 1) // grain) * grain
    while epad % BE != 0:
        epad += grain
    n_nb = npad // BN
    n_eb = epad // BE
    per_w = epad // NUM_WORKERS
    nch = per_w // SC_CHUNK

    WqT = Wq.T
    WkT = Wk.T
    WvT = Wv.T
    WoT = Wo.T
    WhiT = We1[:, :D].T
    WhjT = We1[:, D:2 * D].T
    WlatT = jnp.pad(We1[:, 2 * D:2 * D + 9].T, ((0, 7), (0, 0)))
    WfdT = jnp.pad(We1[:, 2 * D + 9:2 * D + 12].T, ((0, 13), (0, 0)))
    We2T = We2.T
    WnhT = Wn1[:, :D].T
    WnaT = Wn1[:, D:].T
    Wn2T = Wn2.T
    b_row = lambda v: v.reshape(1, -1)

    h0p = jnp.pad(node_features, ((0, npad - N), (0, 0)))
    n2g3 = jnp.pad(node2graph, (0, npad - N)).reshape(n_nb, 1, BN)
    fr128 = jnp.pad(frac_coords, ((0, npad - N), (0, D - 3)))
    lat16 = jnp.pad(lattices.reshape(B, 9), ((0, 0), (0, 7)))
    zrows = jnp.zeros((SC_CHUNK, D), F32)
    orows = jnp.concatenate([jnp.ones((SC_CHUNK, 1), F32),
                             jnp.zeros((SC_CHUNK, D - 1), F32)], axis=1)
    condf = cond_tokens.reshape(B * T, D)

    e0p = jnp.pad(edges[0], (0, epad - E), constant_values=N)
    e1p = jnp.pad(edges[1], (0, epad - E), constant_values=0)
    eg3 = jnp.pad(edge2graph, (0, epad - E),
                  constant_values=0).reshape(n_eb, 1, BE)

    K, V, LP = pl.pallas_call(
        _stage1_body,
        out_shape=(jax.ShapeDtypeStruct((B * T, D), F32),
                   jax.ShapeDtypeStruct((B * T, D), F32),
                   jax.ShapeDtypeStruct((B, D), F32)),
    )(condf, lat16, WkT, b_row(bk), WvT, b_row(bv), WlatT, b_row(be1))
    Kr = K.reshape(B, T * D)
    Vr = V.reshape(B, T * D)

    body2 = functools.partial(_stage2_body, bn=BN, B=B, T=T, D=D)
    h, P1a, P2a = pl.pallas_call(
        body2,
        grid=(n_nb,),
        in_specs=[
            pl.BlockSpec((BN, D), lambda i: (i, 0)),
            pl.BlockSpec((1, 1, BN), lambda i: (i, 0, 0)),
            _const_spec((B, T * D)),
            _const_spec((B, T * D)),
            _const_spec((D, D)),
            _const_spec((1, D)),
            _const_spec((D, D)),
            _const_spec((1, D)),
            _const_spec((D, D)),
            _const_spec((D, D)),
        ],
        out_specs=[
            pl.BlockSpec((BN, D), lambda i: (i, 0)),
            pl.BlockSpec((BN, D), lambda i: (i, 0)),
            pl.BlockSpec((BN, D), lambda i: (i, 0)),
        ],
        out_shape=(jax.ShapeDtypeStruct((npad, D), F32),
                   jax.ShapeDtypeStruct((npad, D), F32),
                   jax.ShapeDtypeStruct((npad, D), F32)),
    )(h0p, n2g3, Kr, Vr, WqT, b_row(bq), WoT, b_row(bo), WhiT, WhjT)

    mesh = plsc.VectorSubcoreMesh(core_axis_name="c", subcore_axis_name="s",
                                  num_cores=2, num_subcores=16)
    count_body = functools.partial(_sc_count_body, per_w=per_w, nch=nch,
                                   ch=SC_CHUNK, npad=npad)
    cnts = pl.kernel(
        count_body,
        out_type=jax.ShapeDtypeStruct((2, npad, D), F32),
        mesh=mesh,
        scratch_types=[
            pltpu.VMEM((SC_CHUNK,), I32),
            pltpu.VMEM((SC_CHUNK, D), F32),
            pltpu.VMEM((SC_CHUNK, D), F32),
            pltpu.VMEM_SHARED((npad, D), F32),
        ],
    )(e0p, orows, zrows)

    gather_body = functools.partial(_sc_gather_body, per_w=per_w, nch=nch,
                                    ch=SC_CHUNK)
    z1, z2, fdh = pl.kernel(
        gather_body,
        out_type=(jax.ShapeDtypeStruct((epad, D), F32),
                  jax.ShapeDtypeStruct((epad, D), F32),
                  jax.ShapeDtypeStruct((epad, 16), F32)),
        mesh=mesh,
        scratch_types=[
            pltpu.VMEM((SC_CHUNK,), I32),
            pltpu.VMEM((SC_CHUNK,), I32),
            pltpu.VMEM((SC_CHUNK, D), F32),
            pltpu.VMEM((SC_CHUNK, D), F32),
            pltpu.VMEM((SC_CHUNK, D), F32),
            pltpu.VMEM((SC_CHUNK, D), F32),
            pltpu.VMEM((SC_CHUNK,), I32),
            pltpu.VMEM((SC_CHUNK,), I32),
            pltpu.VMEM((SC_CHUNK, D), F32),
            pltpu.VMEM((SC_CHUNK, D), F32),
            pltpu.VMEM((SC_CHUNK, D), F32),
            pltpu.VMEM((SC_CHUNK, D), F32),
            pltpu.VMEM((SC_CHUNK, 16), F32),
            pltpu.SemaphoreType.DMA,
            pltpu.SemaphoreType.DMA,
        ],
    )(P1a, P2a, fr128, e0p, e1p)

    body4 = functools.partial(_stage4_body, be=BE, B=B, D=D)
    y = pl.pallas_call(
        body4,
        grid=(n_eb,),
        in_specs=[
            pl.BlockSpec((BE, D), lambda i: (i, 0)),
            pl.BlockSpec((BE, D), lambda i: (i, 0)),
            pl.BlockSpec((BE, 16), lambda i: (i, 0)),
            pl.BlockSpec((1, 1, BE), lambda i: (i, 0, 0)),
            _const_spec((B, D)),
            _const_spec((16, D)),
            _const_spec((D, D)),
            _const_spec((1, D)),
        ],
        out_specs=pl.BlockSpec((BE, D), lambda i: (i, 0)),
        out_shape=jax.ShapeDtypeStruct((epad, D), F32),
    )(z1, z2, fdh, eg3, LP, WfdT, We2T, b_row(be2))

    sum_body = functools.partial(_sc_sum_body, per_w=per_w, nch=nch,
                                 ch=SC_CHUNK, npad=npad)
    sums = pl.kernel(
        sum_body,
        out_type=jax.ShapeDtypeStruct((2, npad, D), F32),
        mesh=mesh,
        scratch_types=[
            pltpu.VMEM((SC_CHUNK,), I32),
            pltpu.VMEM((SC_CHUNK, D), F32),
            pltpu.VMEM((SC_CHUNK,), I32),
            pltpu.VMEM((SC_CHUNK, D), F32),
            pltpu.VMEM((SC_CHUNK, D), F32),
            pltpu.VMEM_SHARED((npad, D), F32),
            pltpu.SemaphoreType.DMA,
            pltpu.SemaphoreType.DMA,
        ],
    )(y, e0p, zrows)

    out = pl.pallas_call(
        _stage6_body,
        grid=(n_nb,),
        in_specs=[
            pl.BlockSpec((BN, D), lambda i: (i, 0)),
            pl.BlockSpec((BN, D), lambda i: (i, 0)),
            pl.BlockSpec((BN, D), lambda i: (i, 0)),
            pl.BlockSpec((BN, D), lambda i: (i, 0)),
            pl.BlockSpec((BN, D), lambda i: (i, 0)),
            pl.BlockSpec((BN, D), lambda i: (i, 0)),
            _const_spec((D, D)),
            _const_spec((D, D)),
            _const_spec((1, D)),
            _const_spec((D, D)),
            _const_spec((1, D)),
        ],
        out_specs=pl.BlockSpec((BN, D), lambda i: (i, 0)),
        out_shape=jax.ShapeDtypeStruct((npad, D), F32),
    )(h, h0p, sums[0], sums[1], cnts[0], cnts[1],
      WnhT, WnaT, b_row(bn1), Wn2T, b_row(bn2))

    return out[:N]
